# Initial kernel scaffold; baseline (speedup 1.0000x reference)
import jax
import jax.numpy as jnp
from jax.experimental import pallas as pl


def kernel(nodes, edge_indices, edge_attr, global_attr, num_nodes, num_edges, batch_indices, W_msg, b_msg, W_upd, b_upd, W_glob, b_glob, W_emb, b_emb):
    raise NotImplementedError("write your pallas kernel here")


# speedup vs baseline: 2.2782x; 2.2782x over previous
import functools

import jax
import jax.numpy as jnp
from jax import lax
from jax.experimental import pallas as pl
from jax.experimental.pallas import tpu as pltpu
from jax.experimental.pallas import tpu_sc as plsc

NEG = -3.0e38
NC, NS, LANES = 2, 16, 16
NW = NC * NS


def _proj_t_body(w_ref, x_ref, out_ref):
    out_ref[...] = lax.dot_general(
        w_ref[...], x_ref[...], (((0,), (1,)), ((), ())),
        preferred_element_type=jnp.float32)


def _proj_t(w, x, block):
    r, k = x.shape
    f = w.shape[1]
    grid = r // block
    return pl.pallas_call(
        _proj_t_body,
        grid=(grid,),
        in_specs=[
            pl.BlockSpec((k, f), lambda i: (0, 0)),
            pl.BlockSpec((block, k), lambda i: (i, 0)),
        ],
        out_specs=pl.BlockSpec((f, block), lambda i: (0, i)),
        out_shape=jax.ShapeDtypeStruct((f, r), jnp.float32),
    )(w, x)


def _make_sc_edge_kernel(n, e, k_chunk):
    nch = e // k_chunk
    assert e % k_chunk == 0 and nch % 2 == 0 and k_chunk % LANES == 0
    steps = k_chunk // LANES
    mesh = plsc.VectorSubcoreMesh(core_axis_name="c", subcore_axis_name="s",
                                  num_cores=NC, num_subcores=NS)

    @functools.partial(
        pl.kernel,
        out_type=jax.ShapeDtypeStruct((NW * n,), jnp.float32),
        mesh=mesh,
        compiler_params=pltpu.CompilerParams(needs_layout_passes=False),
        scratch_types=[
            pltpu.VMEM((n,), jnp.float32),
            pltpu.VMEM((n,), jnp.float32),
            pltpu.VMEM((k_chunk,), jnp.int32),
            pltpu.VMEM((k_chunk,), jnp.int32),
            pltpu.VMEM((k_chunk,), jnp.int32),
            pltpu.VMEM((k_chunk,), jnp.int32),
            pltpu.VMEM((k_chunk,), jnp.float32),
            pltpu.VMEM((k_chunk,), jnp.float32),
            pltpu.SemaphoreType.DMA,
            pltpu.SemaphoreType.DMA,
            pltpu.SemaphoreType.DMA,
            pltpu.SemaphoreType.DMA,
            pltpu.SemaphoreType.DMA,
            pltpu.SemaphoreType.DMA,
        ],
    )
    def sc_kernel(p1t, src, tgt, ea2t, out,
                  p1_v, acc_v, sb0, sb1, tb0, tb1, eb0, eb1,
                  ss0, ss1, ts0, ts1, es0, es1):
        wid = lax.axis_index("s") * NC + lax.axis_index("c")
        src_bufs, tgt_bufs, ea_bufs = (sb0, sb1), (tb0, tb1), (eb0, eb1)
        src_sems, tgt_sems, ea_sems = (ss0, ss1), (ts0, ts1), (es0, es1)

        pltpu.sync_copy(p1t.at[pl.ds(pl.multiple_of(wid * n, 8), n)], p1_v)

        def init_body(i, c):
            acc_v[pl.ds(pl.multiple_of(i * LANES, LANES), LANES)] = jnp.full(
                (LANES,), NEG, jnp.float32)
            return c
        lax.fori_loop(0, n // LANES, init_body, 0)

        def chunk_copies(c, bi):
            off = pl.multiple_of(c * k_chunk, 8)
            return (
                pltpu.make_async_copy(
                    src.at[pl.ds(off, k_chunk)], src_bufs[bi], src_sems[bi]),
                pltpu.make_async_copy(
                    tgt.at[pl.ds(off, k_chunk)], tgt_bufs[bi], tgt_sems[bi]),
                pltpu.make_async_copy(
                    ea2t.at[pl.ds(pl.multiple_of(wid * e + off, 8), k_chunk)],
                    ea_bufs[bi], ea_sems[bi]),
            )

        def start_chunk(c, bi):
            for cp in chunk_copies(c, bi):
                cp.start()

        def wait_chunk(c, bi):
            for cp in chunk_copies(c, bi):
                cp.wait()

        iota = lax.iota(jnp.int32, LANES)
        idx_next = jnp.minimum(iota + 1, LANES - 1)
        last_lane = iota == LANES - 1
        shift_idx = []
        for d in (1, 2, 4, 8):
            shift_idx.append((jnp.maximum(iota - d, 0), iota >= d))

        def make_step(sb, tb, eb):
            def step(i, c):
                base = pl.multiple_of(i * LANES, LANES)
                s_idx = sb[pl.ds(base, LANES)]
                t_idx = tb[pl.ds(base, LANES)]
                ea = eb[pl.ds(base, LANES)]
                val = plsc.load_gather(p1_v, [s_idx]) + ea
                _, last_m = plsc.scan_count(t_idx)
                old = plsc.load_gather(acc_v, [t_idx])
                plsc.store_scatter(acc_v, [t_idx], jnp.maximum(old, val),
                                   mask=last_m)

                @pl.when(jnp.logical_not(jnp.all(last_m)))
                def _slow():
                    ks, vs = plsc.sort_key_val(t_idx, val)
                    for idxd, valid in shift_idx:
                        ksh = ks.at[idxd].get(mode="promise_in_bounds")
                        vsh = vs.at[idxd].get(mode="promise_in_bounds")
                        ok = (ksh == ks) & valid
                        vs = jnp.maximum(vs, jnp.where(ok, vsh, NEG))
                    knx = ks.at[idx_next].get(mode="promise_in_bounds")
                    is_last = (knx != ks) | last_lane
                    old2 = plsc.load_gather(acc_v, [ks])
                    plsc.store_scatter(acc_v, [ks], jnp.maximum(old2, vs),
                                       mask=is_last)
                return c
            return step

        start_chunk(0, 0)

        def outer(c2, carry):
            c0 = c2 * 2
            for b in (0, 1):
                c = c0 + b
                wait_chunk(c, b)

                @pl.when(c + 1 < nch)
                def _():
                    start_chunk(c + 1, 1 - b)

                lax.fori_loop(0, steps,
                              make_step(src_bufs[b], tgt_bufs[b], ea_bufs[b]),
                              carry)
            return carry
        lax.fori_loop(0, nch // 2, outer, 0)

        pltpu.sync_copy(acc_v, out.at[pl.ds(pl.multiple_of(wid * n, 8), n)])

    return sc_kernel


def _post_body(nodes_ref, smx_ref, g_ref, w3_ref, w4_ref, bm_ref,
               wn_ref, wa_ref, wg_ref, bu_ref, we_ref, be_ref, out_ref):
    g = g_ref[...]
    gm = lax.dot_general(w4_ref[...], g, (((0,), (1,)), ((), ())),
                         preferred_element_type=jnp.float32) + bm_ref[...]
    p3t = lax.dot_general(w3_ref[...], nodes_ref[...], (((0,), (1,)), ((), ())),
                          preferred_element_type=jnp.float32)
    aggr_t = jnp.maximum(smx_ref[...] + p3t + gm, 0.0)
    gu = lax.dot_general(g, wg_ref[...], (((1,), (0,)), ((), ())),
                         preferred_element_type=jnp.float32) + bu_ref[...]
    upd = lax.dot_general(nodes_ref[...], wn_ref[...], (((1,), (0,)), ((), ())),
                          preferred_element_type=jnp.float32)
    upd += lax.dot_general(aggr_t, wa_ref[...], (((0,), (0,)), ((), ())),
                           preferred_element_type=jnp.float32)
    upd = jnp.maximum(upd + gu, 0.0)
    out = lax.dot_general(upd, we_ref[...], (((1,), (0,)), ((), ())),
                          preferred_element_type=jnp.float32)
    out_ref[...] = jnp.maximum(out + be_ref[...], 0.0)


def _post(nodes, smx_t, g, w3, w4, bm_col, wn, wa, wg, bu, we, be, block):
    n, df = nodes.shape
    dm = w3.shape[1]
    du = wn.shape[1]
    dout = we.shape[1]
    dg = g.shape[1]
    grid = n // block
    full = lambda *shape: pl.BlockSpec(shape, lambda i: tuple(0 for _ in shape))
    return pl.pallas_call(
        _post_body,
        grid=(grid,),
        in_specs=[
            pl.BlockSpec((block, df), lambda i: (i, 0)),
            pl.BlockSpec((dm, block), lambda i: (0, i)),
            full(1, dg), full(df, dm), full(dg, dm), full(dm, 1),
            full(df, du), full(dm, du), full(dg, du), full(du,),
            full(du, dout), full(dout,),
        ],
        out_specs=pl.BlockSpec((block, dout), lambda i: (i, 0)),
        out_shape=jax.ShapeDtypeStruct((n, dout), jnp.float32),
    )(nodes, smx_t, g, w3, w4, bm_col, wn, wa, wg, bu, we, be)


def kernel(nodes, edge_indices, edge_attr, global_attr, num_nodes, num_edges,
           batch_indices, W_msg, b_msg, W_upd, b_upd, W_glob, b_glob,
           W_emb, b_emb):
    n, df = nodes.shape
    e, de = edge_attr.shape
    dm = W_msg.shape[1]
    src = edge_indices[0, :, 0].astype(jnp.int32)
    tgt = edge_indices[0, :, 1].astype(jnp.int32)

    w1 = W_msg[:df]
    w2 = W_msg[df:df + de]
    w3 = W_msg[df + de:df + de + df]
    w4 = W_msg[df + de + df:]

    p1t = _proj_t(w1, nodes, block=n)
    ea2t = _proj_t(w2, edge_attr, block=16000)

    sc = _make_sc_edge_kernel(n, e, k_chunk=8000)
    smx_t = sc(p1t.reshape(-1), src, tgt, ea2t.reshape(-1)).reshape(dm, n)

    wn = W_upd[:df]
    wa = W_upd[df:df + dm]
    wg = W_upd[df + dm:]
    bm_col = b_msg.reshape(dm, 1)
    return _post(nodes, smx_t, global_attr, w3, w4, bm_col,
                 wn, wa, wg, b_upd, W_emb, b_emb, block=n)
